# Initial kernel scaffold; baseline (speedup 1.0000x reference)
import jax
import jax.numpy as jnp
from jax.experimental import pallas as pl


def kernel(x, edge_index, batch, block_embeddings, block_indices_for_reaction, W1, b1, W2, b2, W3, b3, Wr1, br1, Wr2, br2, temperature):
    raise NotImplementedError("write your pallas kernel here")


# speedup vs baseline: 7.5924x; 7.5924x over previous
import functools

import jax
import jax.numpy as jnp
from jax import lax
from jax.experimental import pallas as pl
from jax.experimental.pallas import tpu as pltpu
from jax.experimental.pallas import tpu_sc as plsc

N_NODES = 10000
D = 128
B_GRAPHS = 256
N_TEMPLATES = 512

NC = 2
NS = 16
NW = NC * NS
LANE = 128
CH = 79
EPAD = NW * CH * LANE
TRASH = N_NODES
ACC_R = 10112
RPT = ACC_R // NS

NHALF = 5056
CACC = 5120
LTRASH = NHALF
CH2 = 157
EPAD2 = NS * CH2 * LANE
RPT2 = CACC // NS

POOL_CH = 3
POOL_N = NW * POOL_CH * LANE
POOL_R = 264

@functools.lru_cache(maxsize=None)
def _get_mesh():
    return plsc.VectorSubcoreMesh(core_axis_name="c", subcore_axis_name="s",
                                  num_cores=NC, num_subcores=NS)


def _fill(ref, rows, value):
    w = ref.shape[1]

    def body(i, carry):
        for j in range(w // 16):
            ref[i, pl.ds(j * 16, 16)] = jnp.full((16,), value, jnp.float32)
        return carry

    lax.fori_loop(0, rows, body, 0)


@functools.lru_cache(maxsize=None)
def _make_deg_sc():
    return pl.kernel(
        _deg_body,
        out_type=jax.ShapeDtypeStruct((NC, ACC_R, 16), jnp.float32),
        mesh=_get_mesh(),
        scratch_types=[
            pltpu.VMEM((CH, LANE), jnp.int32),
            pltpu.VMEM((LANE, 16), jnp.float32),
            pltpu.VMEM((LANE, 16), jnp.float32),
            pltpu.VMEM_SHARED((ACC_R, 16), jnp.float32),
        ],
    )


def _deg_body(dst_hbm, out_hbm, dst_v, ones_v, z_v, acc_sh):
    c = lax.axis_index("c")
    s = lax.axis_index("s")
    wid = s * NC + c
    _fill(ones_v, LANE, 1.0)
    _fill(z_v, LANE, 0.0)

    base = s * RPT
    off = 0
    for sz in (128, 128, 128, 128, 120):
        pltpu.sync_copy(z_v.at[pl.ds(0, sz)], acc_sh.at[pl.ds(base + off, sz)])
        off += sz
    plsc.subcore_barrier()
    pltpu.sync_copy(dst_hbm.at[wid], dst_v)

    def body(j, carry):
        pltpu.sync_copy(ones_v, acc_sh.at[dst_v.at[j]], add=True)
        return carry

    lax.fori_loop(0, CH, body, 0)
    plsc.subcore_barrier()
    pltpu.sync_copy(acc_sh.at[pl.ds(base, RPT)], out_hbm.at[c].at[pl.ds(base, RPT)])


@functools.lru_cache(maxsize=None)
def _make_agg_sc():
    return pl.kernel(
        _agg_body,
        out_type=jax.ShapeDtypeStruct((NC, CACC, D), jnp.float32),
        mesh=_get_mesh(),
        scratch_types=[
            pltpu.VMEM((CH2, LANE), jnp.int32),
            pltpu.VMEM((CH2, LANE), jnp.int32),
            pltpu.VMEM((LANE, D), jnp.float32),
            pltpu.VMEM((LANE, D), jnp.float32),
            pltpu.VMEM_SHARED((CACC, D), jnp.float32),
            pltpu.SemaphoreType.DMA,
        ],
    )


def _agg_body(h2_hbm, src_hbm, dstl_hbm, out_hbm, src_v, dst_v, msg_v, z_v,
              acc_sh, sem):
    c = lax.axis_index("c")
    s = lax.axis_index("s")
    _fill(z_v, LANE, 0.0)

    base = s * RPT2
    off = 0
    for sz in (128, 128, 64):
        pltpu.sync_copy(z_v.at[pl.ds(0, sz)], acc_sh.at[pl.ds(base + off, sz)])
        off += sz
    plsc.subcore_barrier()
    pltpu.sync_copy(src_hbm.at[s], src_v)
    pltpu.sync_copy(dstl_hbm.at[c].at[s], dst_v)

    def body(j, carry):
        pltpu.async_copy(h2_hbm.at[src_v.at[j]], msg_v, sem).wait()
        pltpu.sync_copy(msg_v, acc_sh.at[dst_v.at[j]], add=True)
        return carry

    lax.fori_loop(0, CH2, body, 0)
    plsc.subcore_barrier()
    pltpu.sync_copy(acc_sh.at[pl.ds(base, RPT2)], out_hbm.at[c].at[pl.ds(base, RPT2)])


@functools.lru_cache(maxsize=None)
def _make_pool_sc():
    return pl.kernel(
        _pool_body,
        out_type=(
            jax.ShapeDtypeStruct((NC, POOL_R, D), jnp.float32),
            jax.ShapeDtypeStruct((NC, POOL_R, 16), jnp.float32),
            jax.ShapeDtypeStruct((B_GRAPHS, D), jnp.float32),
        ),
        mesh=_get_mesh(),
        scratch_types=[
            pltpu.VMEM((POOL_CH, LANE), jnp.int32),
            pltpu.VMEM((POOL_CH * LANE, D), jnp.float32),
            pltpu.VMEM((LANE, D), jnp.float32),
            pltpu.VMEM((LANE, 16), jnp.float32),
            pltpu.VMEM((LANE, 16), jnp.float32),
            pltpu.VMEM((1, LANE), jnp.int32),
            pltpu.VMEM_SHARED((POOL_R, D), jnp.float32),
            pltpu.VMEM_SHARED((POOL_R, 16), jnp.float32),
            pltpu.SemaphoreType.DMA,
        ],
    )


def _pool_body(g_hbm, batch_hbm, selidx_hbm, emb_hbm,
               sums_hbm, cnts_hbm, sel_hbm,
               idx_v, rows_v, zd_v, ones_v, z16_v, si_v, acc_sh, accc_sh, sem):
    c = lax.axis_index("c")
    s = lax.axis_index("s")
    wid = s * NC + c
    _fill(ones_v, LANE, 1.0)
    _fill(z16_v, LANE, 0.0)
    _fill(zd_v, LANE, 0.0)

    @pl.when(s == 0)
    def _zero():
        off = 0
        for sz in (128, 128, 8):
            pltpu.sync_copy(zd_v.at[pl.ds(0, sz)], acc_sh.at[pl.ds(off, sz)])
            pltpu.sync_copy(z16_v.at[pl.ds(0, sz)], accc_sh.at[pl.ds(off, sz)])
            off += sz

    plsc.subcore_barrier()
    pltpu.sync_copy(g_hbm.at[pl.ds(wid * POOL_CH * LANE, POOL_CH * LANE)], rows_v)
    pltpu.sync_copy(batch_hbm.at[wid], idx_v)
    for j in range(POOL_CH):
        pltpu.sync_copy(rows_v.at[pl.ds(j * LANE, LANE)],
                        acc_sh.at[idx_v.at[j]], add=True)
        pltpu.sync_copy(ones_v, accc_sh.at[idx_v.at[j]], add=True)
    plsc.subcore_barrier()

    @pl.when(s == 0)
    def _out():
        pltpu.sync_copy(acc_sh, sums_hbm.at[c])
        pltpu.sync_copy(accc_sh, cnts_hbm.at[c])

    @pl.when(s == 1)
    def _sel():
        pltpu.sync_copy(selidx_hbm.at[pl.ds(c, 1)], si_v)
        pltpu.async_copy(emb_hbm.at[si_v.at[0]], zd_v, sem).wait()
        pltpu.sync_copy(zd_v, sel_hbm.at[pl.ds(c * LANE, LANE)])


_BLK = 1024


def _dinv_of(d0, d1):
    return 1.0 / jnp.sqrt(1.0 + d0[:, 0:1] + d1[:, 0:1])


def _m1_body(x_ref, w_ref, d0_ref, d1_ref, o_ref):
    dinv = _dinv_of(d0_ref[...], d1_ref[...])
    o_ref[...] = jnp.dot(x_ref[...], w_ref[...],
                         preferred_element_type=jnp.float32) * dinv


def _fused_body(a_ref, p_ref, d0_ref, d1_ref, b_ref, w_ref, o_ref):
    dinv = _dinv_of(d0_ref[...], d1_ref[...])
    g = jnp.maximum(dinv * (a_ref[...] + p_ref[...]) + b_ref[...], 0.0)
    o_ref[...] = jnp.dot(g, w_ref[...], preferred_element_type=jnp.float32) * dinv


def _g3_body(a_ref, p_ref, d0_ref, d1_ref, b_ref, o_ref):
    dinv = _dinv_of(d0_ref[...], d1_ref[...])
    o_ref[...] = jnp.maximum(
        dinv * (a_ref[...] + p_ref[...]) + b_ref[...], 0.0)


def _remap_body(d_ref, a_ref, b_ref):
    d = d_ref[...]
    a_ref[...] = jnp.where(d < NHALF, d, LTRASH)
    b_ref[...] = jnp.where((d >= NHALF) & (d < N_NODES), d - NHALF, LTRASH)


def _head_body(s0_ref, s1_ref, c0_ref, c1_ref, sel_ref, wr1_ref, br1_ref,
               wr2_ref, br2_ref, t_ref, hn_ref, rxn_ref):
    sums = s0_ref[...] + s1_ref[...]
    cnt = c0_ref[:, 0:1] + c1_ref[:, 0:1]
    h_state = sums / jnp.maximum(cnt, 1.0)
    n = jnp.sqrt(jnp.sum(h_state * h_state, axis=-1, keepdims=True))
    hn = h_state / jnp.maximum(n, 1e-12)
    temp = jnp.maximum(t_ref[0, 0], 1e-4)
    hn_ref[...] = hn / temp
    sel = sel_ref[...]
    rxn_in = jnp.concatenate([h_state, sel], axis=1)
    t = jnp.maximum(
        jnp.dot(rxn_in, wr1_ref[...], preferred_element_type=jnp.float32)
        + br1_ref[...], 0.0)
    rxn_ref[...] = jnp.dot(t, wr2_ref[...],
                           preferred_element_type=jnp.float32) + br2_ref[...]


_EBLK = 2048


def _logits_body(hn_ref, e_ref, o_ref):
    e = e_ref[...]
    n = jnp.sqrt(jnp.sum(e * e, axis=-1, keepdims=True))
    en = e / jnp.maximum(n, 1e-12)
    o_ref[...] = lax.dot_general(hn_ref[...], en, (((1,), (1,)), ((), ())),
                                 preferred_element_type=jnp.float32)


def _row_grid(body, n_rows, in_specs, out_shape):
    grid = (pl.cdiv(n_rows, _BLK),)
    return pl.pallas_call(body, grid=grid, in_specs=in_specs,
                          out_specs=pl.BlockSpec((_BLK, out_shape.shape[1]),
                                                 lambda i: (i, 0)),
                          out_shape=out_shape)


def kernel(x, edge_index, batch, block_embeddings, block_indices_for_reaction,
           W1, b1, W2, b2, W3, b3, Wr1, br1, Wr2, br2, temperature):
    f32 = jnp.float32
    src = edge_index[0].astype(jnp.int32)
    dst = edge_index[1].astype(jnp.int32)
    pad = EPAD - src.shape[0]
    dst_p = jnp.concatenate([dst, jnp.full((pad,), TRASH, jnp.int32)]).reshape(NW, CH, LANE)
    pad2 = EPAD2 - src.shape[0]
    src2 = jnp.concatenate([src, jnp.zeros((pad2,), jnp.int32)]).reshape(NS, CH2, LANE)
    dst2f = jnp.concatenate([dst, jnp.full((pad2,), TRASH, jnp.int32)]).reshape(NS * CH2, LANE)
    batch_p = jnp.concatenate([
        batch.astype(jnp.int32),
        jnp.full((POOL_N - batch.shape[0],), B_GRAPHS, jnp.int32)
    ]).reshape(NW, POOL_CH, LANE)
    sel_idx = block_indices_for_reaction.astype(jnp.int32).reshape(NC, LANE)

    deg = _make_deg_sc()(dst_p)
    d0 = deg[0, :N_NODES]
    d1 = deg[1, :N_NODES]

    row = pl.BlockSpec((_BLK, D), lambda i: (i, 0))
    row16 = pl.BlockSpec((_BLK, 16), lambda i: (i, 0))
    wfull = pl.BlockSpec((D, D), lambda i: (0, 0))
    brow = pl.BlockSpec((1, D), lambda i: (0, 0))
    b1r, b2r, b3r = b1.reshape(1, D), b2.reshape(1, D), b3.reshape(1, D)

    h2 = _row_grid(_m1_body, N_NODES, [row, wfull, row16, row16],
                   jax.ShapeDtypeStruct((N_NODES, D), f32))(x, W1, d0, d1)

    rowi = pl.BlockSpec((512, LANE), lambda i: (i, 0))
    dstA, dstB = pl.pallas_call(
        _remap_body, grid=(pl.cdiv(NS * CH2, 512),),
        in_specs=[rowi], out_specs=(rowi, rowi),
        out_shape=(jax.ShapeDtypeStruct((NS * CH2, LANE), jnp.int32),
                   jax.ShapeDtypeStruct((NS * CH2, LANE), jnp.int32)),
    )(dst2f)
    dstl = jnp.stack([dstA.reshape(NS, CH2, LANE), dstB.reshape(NS, CH2, LANE)])

    def assemble(a):
        return jnp.concatenate([a[0, :NHALF], a[1, :N_NODES - NHALF]], axis=0)

    Ws = jnp.stack([W2, W3, W2])
    bs = jnp.stack([b1r, b2r, b3r])
    agg_sc = _make_agg_sc()
    fused = _row_grid(_fused_body, N_NODES,
                      [row, row, row16, row16, brow, wfull],
                      jax.ShapeDtypeStruct((N_NODES, D), f32))

    def layer_body(k, carry):
        h2_cur, _, _ = carry
        a = agg_sc(h2_cur, src2, dstl)
        h2n = fused(assemble(a), h2_cur, d0, d1, bs[k], Ws[k])
        return (h2n, h2_cur, a)

    init = (h2, h2, jnp.zeros((NC, CACC, D), f32))
    _, h2, agg = lax.fori_loop(0, 3, layer_body, init)
    rowc = pl.BlockSpec((_BLK, D), lambda i: (jnp.minimum(i, N_NODES // _BLK), 0))
    rowc16 = pl.BlockSpec((_BLK, 16), lambda i: (jnp.minimum(i, N_NODES // _BLK), 0))
    browc = pl.BlockSpec((1, D), lambda i: (0, 0))
    g3 = pl.pallas_call(
        _g3_body, grid=(POOL_N // _BLK,),
        in_specs=[rowc, rowc, rowc16, rowc16, browc],
        out_specs=pl.BlockSpec((_BLK, D), lambda i: (i, 0)),
        out_shape=jax.ShapeDtypeStruct((POOL_N, D), f32),
    )(assemble(agg), h2, d0, d1, b3r)

    sums, cnts, selected = _make_pool_sc()(g3, batch_p, sel_idx, block_embeddings)

    full = lambda r, c_: pl.BlockSpec((r, c_), lambda: (0, 0))
    hn_s, rxn_logits = pl.pallas_call(
        _head_body,
        in_specs=[full(B_GRAPHS, D), full(B_GRAPHS, D), full(B_GRAPHS, 16),
                  full(B_GRAPHS, 16), full(B_GRAPHS, D), full(2 * D, D),
                  full(1, D), full(D, N_TEMPLATES), full(1, N_TEMPLATES),
                  full(1, 1)],
        out_specs=(full(B_GRAPHS, D), full(B_GRAPHS, N_TEMPLATES)),
        out_shape=(jax.ShapeDtypeStruct((B_GRAPHS, D), f32),
                   jax.ShapeDtypeStruct((B_GRAPHS, N_TEMPLATES), f32)),
    )(sums[0, :B_GRAPHS], sums[1, :B_GRAPHS], cnts[0, :B_GRAPHS],
      cnts[1, :B_GRAPHS], selected, Wr1, br1.reshape(1, D), Wr2,
      br2.reshape(1, N_TEMPLATES), temperature.reshape(1, 1).astype(f32))

    n_blocks = block_embeddings.shape[0]
    block_logits = pl.pallas_call(
        _logits_body, grid=(pl.cdiv(n_blocks, _EBLK),),
        in_specs=[pl.BlockSpec((B_GRAPHS, D), lambda i: (0, 0)),
                  pl.BlockSpec((_EBLK, D), lambda i: (i, 0))],
        out_specs=pl.BlockSpec((B_GRAPHS, _EBLK), lambda i: (0, i)),
        out_shape=jax.ShapeDtypeStruct((B_GRAPHS, n_blocks), f32),
    )(hn_s, block_embeddings)

    return (block_logits, rxn_logits)
